# Initial kernel scaffold; baseline (speedup 1.0000x reference)
import jax
import jax.numpy as jnp
from jax.experimental import pallas as pl


def kernel(prediction, target):
    raise NotImplementedError("write your pallas kernel here")


# speedup vs baseline: 36.8620x; 36.8620x over previous
import functools

import jax
import jax.numpy as jnp
from jax import lax
from jax.experimental import pallas as pl
from jax.experimental.pallas import tpu as pltpu
from jax.experimental.pallas import tpu_sc as plsc

_BINS = 100
_L = 16
_NC = 2
_NS = 16
_NW = _NC * _NS

_N = 32 * 3 * 512 * 512
_PER_TILE = _N // _NW
_CHUNK = 65536
_NCH = _PER_TILE // _CHUNK
_UNROLL = 8
_INNER = _CHUNK // (_L * _UNROLL)

_mesh = plsc.VectorSubcoreMesh(core_axis_name="c", subcore_axis_name="s")
_cparams = pltpu.CompilerParams(needs_layout_passes=False)


def _wid():
    return lax.axis_index("s") * _NC + lax.axis_index("c")


@functools.partial(
    pl.kernel,
    mesh=_mesh,
    out_type=[
        jax.ShapeDtypeStruct((_NW, _L), jnp.float32),
        jax.ShapeDtypeStruct((_NW, _L), jnp.float32),
    ],
    scratch_types=[
        pltpu.VMEM((_CHUNK,), jnp.float32),
        pltpu.VMEM((_L,), jnp.float32),
        pltpu.VMEM((_L,), jnp.float32),
    ],
    compiler_params=_cparams,
)
def _minmax_k(p_hbm, t_hbm, mn_out, mx_out, buf, mnv, mxv):
    base = _wid() * _PER_TILE

    def one_tensor(src, carry):
        def chunk_body(g, carry):
            pltpu.sync_copy(src.at[pl.ds(base + g * _CHUNK, _CHUNK)], buf)

            def vec_body(i, carry):
                mn, mx = carry
                off = i * (_L * _UNROLL)
                for u in range(_UNROLL):
                    x = buf[pl.ds(off + u * _L, _L)]
                    mn = jnp.minimum(mn, x)
                    mx = jnp.maximum(mx, x)
                return mn, mx

            return lax.fori_loop(0, _INNER, vec_body, carry)

        return lax.fori_loop(0, _NCH, chunk_body, carry)

    mn = jnp.full((_L,), jnp.inf, jnp.float32)
    mx = jnp.full((_L,), -jnp.inf, jnp.float32)
    mn, mx = one_tensor(p_hbm, (mn, mx))
    mn, mx = one_tensor(t_hbm, (mn, mx))
    mnv[...] = mn
    mxv[...] = mx
    w = _wid()
    pltpu.sync_copy(mnv, mn_out.at[w])
    pltpu.sync_copy(mxv, mx_out.at[w])


@functools.partial(
    pl.kernel,
    mesh=_mesh,
    out_type=jax.ShapeDtypeStruct((_NW, _BINS * _L), jnp.float32),
    scratch_types=[
        pltpu.VMEM((_CHUNK,), jnp.float32),
        pltpu.VMEM((_BINS * _L,), jnp.float32),
        pltpu.VMEM((2 * _L,), jnp.float32),
    ],
    compiler_params=_cparams,
)
def _hist_k(p_hbm, t_hbm, par_hbm, out, buf, hist, par):
    w = _wid()
    base = w * _PER_TILE
    pltpu.sync_copy(par_hbm, par)
    lo = par[pl.ds(0, _L)]
    width = par[pl.ds(_L, _L)]
    lane = lax.iota(jnp.int32, _L)

    def zero_body(i, _):
        hist[pl.ds(i * _L, _L)] = jnp.zeros((_L,), jnp.float32)
        return 0

    lax.fori_loop(0, _BINS, zero_body, 0)

    def one_tensor(src, sign):
        valv = jnp.full((_L,), sign, jnp.float32)

        def chunk_body(g, _):
            pltpu.sync_copy(src.at[pl.ds(base + g * _CHUNK, _CHUNK)], buf)

            def vec_body(i, _):
                off = i * (_L * _UNROLL)
                for u in range(_UNROLL):
                    x = buf[pl.ds(off + u * _L, _L)]
                    scaled = (x - lo) / width * 100.0
                    idx = scaled.astype(jnp.int32)
                    idx = jnp.minimum(idx, _BINS - 1)
                    idx = jnp.maximum(idx, 0)
                    addr = idx * _L + lane
                    plsc.addupdate_scatter(hist, [addr], valv, mask=x >= lo)
                return 0

            lax.fori_loop(0, _INNER, vec_body, 0)
            return 0

        lax.fori_loop(0, _NCH, chunk_body, 0)

    one_tensor(p_hbm, 1.0)
    one_tensor(t_hbm, -1.0)
    pltpu.sync_copy(hist, out.at[w])


def kernel(prediction, target):
    p = prediction.reshape(-1)
    t = target.reshape(-1)
    mns, mxs = _minmax_k(p, t)
    lo = jnp.min(mns) + 0.1
    width = jnp.max(mxs) - lo
    params = jnp.concatenate(
        [jnp.full((_L,), lo, jnp.float32), jnp.full((_L,), width, jnp.float32)]
    )
    parts = _hist_k(p, t, params)
    diff = parts.reshape(_NW, _BINS, _L).sum(axis=(0, 2))
    return jnp.mean(jnp.abs(diff))


# speedup vs baseline: 166.5059x; 4.5170x over previous
import functools

import jax
import jax.numpy as jnp
from jax import lax
from jax.experimental import pallas as pl
from jax.experimental.pallas import tpu as pltpu
from jax.experimental.pallas import tpu_sc as plsc

_BINS = 100
_L = 16
_NC = 2
_NS = 16
_NW = _NC * _NS

_P = 512
_PLANES = 96
_PPT = _PLANES // _NW
_CR = 64
_CPP = _P // _CR
_NCH = _PPT * _CPP
_VPR = _P // _L
_GRP = 8

_mesh = plsc.VectorSubcoreMesh(core_axis_name="c", subcore_axis_name="s")
_cparams = pltpu.CompilerParams(
    needs_layout_passes=False, use_tc_tiling_on_sc=True
)


def _wid():
    return lax.axis_index("s") * _NC + lax.axis_index("c")


def _start(src, pb, j, slotbuf, sem):
    plane = pb + (j >> 3)
    r0 = (j & 7) * _CR
    pltpu.async_copy(src.at[plane, pl.ds(r0, _CR)], slotbuf, sem)


def _wait(src, slotbuf, sem):
    pltpu.make_async_copy(src.at[0, pl.ds(0, _CR)], slotbuf, sem).wait()


def _pipeline(src, pb, buf, sem0, sem1, compute, init_carry):
    _start(src, pb, 0, buf.at[0], sem0)

    def pair_body(j, carry):
        _wait(src, buf.at[0], sem0)
        _start(src, pb, 2 * j + 1, buf.at[1], sem1)
        carry = compute(buf.at[0], carry)
        _wait(src, buf.at[1], sem1)

        @pl.when(j < _NCH // 2 - 1)
        def _():
            _start(src, pb, 2 * j + 2, buf.at[0], sem0)

        return compute(buf.at[1], carry)

    return lax.fori_loop(0, _NCH // 2, pair_body, init_carry, unroll=False)


@functools.partial(
    pl.kernel,
    mesh=_mesh,
    out_type=[
        jax.ShapeDtypeStruct((_NW, _L), jnp.float32),
        jax.ShapeDtypeStruct((_NW, _L), jnp.float32),
    ],
    scratch_types=[
        pltpu.VMEM((2, _CR, _P), jnp.float32),
        pltpu.VMEM((_L,), jnp.float32),
        pltpu.VMEM((_L,), jnp.float32),
        pltpu.SemaphoreType.DMA,
        pltpu.SemaphoreType.DMA,
    ],
    compiler_params=_cparams,
)
def _minmax_k(p_hbm, t_hbm, mn_out, mx_out, buf, mnv, mxv, sem0, sem1):
    w = _wid()
    pb = w * _PPT

    def compute(slab, carry):
        def row_body(r, carry):
            mns, mxs = carry
            for g in range(_VPR // _GRP):
                xs = [
                    slab[r, pl.ds((g * _GRP + u) * _L, _L)]
                    for u in range(_GRP)
                ]
                mns = tuple(jnp.minimum(m, x) for m, x in zip(mns, xs))
                mxs = tuple(jnp.maximum(m, x) for m, x in zip(mxs, xs))
            return mns, mxs

        return lax.fori_loop(0, _CR, row_body, carry)

    mns = tuple(jnp.full((_L,), jnp.inf, jnp.float32) for _ in range(_GRP))
    mxs = tuple(jnp.full((_L,), -jnp.inf, jnp.float32) for _ in range(_GRP))
    carry = _pipeline(p_hbm, pb, buf, sem0, sem1, compute, (mns, mxs))
    carry = _pipeline(t_hbm, pb, buf, sem0, sem1, compute, carry)
    mns, mxs = carry
    mn, mx = mns[0], mxs[0]
    for u in range(1, _GRP):
        mn = jnp.minimum(mn, mns[u])
        mx = jnp.maximum(mx, mxs[u])
    mnv[...] = mn
    mxv[...] = mx
    pltpu.sync_copy(mnv, mn_out.at[w])
    pltpu.sync_copy(mxv, mx_out.at[w])


@functools.partial(
    pl.kernel,
    mesh=_mesh,
    out_type=jax.ShapeDtypeStruct((_NW, _BINS * _L), jnp.float32),
    scratch_types=[
        pltpu.VMEM((2, _CR, _P), jnp.float32),
        pltpu.VMEM((_BINS * _L,), jnp.float32),
        pltpu.VMEM((2 * _L,), jnp.float32),
        pltpu.SemaphoreType.DMA,
        pltpu.SemaphoreType.DMA,
    ],
    compiler_params=_cparams,
)
def _hist_k(p_hbm, t_hbm, par_hbm, out, buf, hist, par, sem0, sem1):
    w = _wid()
    pb = w * _PPT
    pltpu.sync_copy(par_hbm, par)
    lo = par[pl.ds(0, _L)]
    width = par[pl.ds(_L, _L)]
    lane = lax.iota(jnp.int32, _L)

    def zero_body(i, _):
        hist[pl.ds(i * _L, _L)] = jnp.zeros((_L,), jnp.float32)
        return 0

    lax.fori_loop(0, _BINS, zero_body, 0)

    def make_compute(sign):
        valv = jnp.full((_L,), sign, jnp.float32)

        def compute(slab, carry):
            def row_body(r, _):
                for g in range(_VPR // _GRP):
                    xs = [
                        slab[r, pl.ds((g * _GRP + u) * _L, _L)]
                        for u in range(_GRP)
                    ]
                    ss = [(x - lo) / width * 100.0 for x in xs]
                    idxs = [s.astype(jnp.int32) for s in ss]
                    idxs = [
                        plsc.bitcast(
                            jnp.minimum(
                                plsc.bitcast(v, jnp.uint32), _BINS - 1
                            ),
                            jnp.int32,
                        )
                        for v in idxs
                    ]
                    addrs = [v * _L + lane for v in idxs]
                    masks = [x >= lo for x in xs]
                    for u in range(_GRP):
                        plsc.addupdate_scatter(
                            hist, [addrs[u]], valv, mask=masks[u]
                        )
                return 0

            lax.fori_loop(0, _CR, row_body, 0)
            return carry

        return compute

    _pipeline(p_hbm, pb, buf, sem0, sem1, make_compute(1.0), 0)
    _pipeline(t_hbm, pb, buf, sem0, sem1, make_compute(-1.0), 0)
    pltpu.sync_copy(hist, out.at[w])


def kernel(prediction, target):
    p = prediction.reshape(_PLANES, _P, _P)
    t = target.reshape(_PLANES, _P, _P)
    mns, mxs = _minmax_k(p, t)
    lo = jnp.min(mns) + 0.1
    width = jnp.max(mxs) - lo
    params = jnp.concatenate(
        [jnp.full((_L,), lo, jnp.float32), jnp.full((_L,), width, jnp.float32)]
    )
    parts = _hist_k(p, t, params)
    diff = parts.reshape(_NW, _BINS, _L).sum(axis=(0, 2))
    return jnp.mean(jnp.abs(diff))


# speedup vs baseline: 239.1664x; 1.4364x over previous
import functools

import jax
import jax.numpy as jnp
from jax import lax
from jax.experimental import pallas as pl
from jax.experimental.pallas import tpu as pltpu
from jax.experimental.pallas import tpu_sc as plsc

_BINS = 100
_L = 16
_NC = 2
_NS = 16
_NW = _NC * _NS

_P = 512
_PLANES = 96
_PPT = _PLANES // _NW
_CR = 64
_CPP = _P // _CR
_NCH = _PPT * _CPP
_VPR = _P // _L
_GRP = 16
_TBINS = _BINS + 1

_mesh = plsc.VectorSubcoreMesh(core_axis_name="c", subcore_axis_name="s")
_cparams = pltpu.CompilerParams(
    needs_layout_passes=False, use_tc_tiling_on_sc=True
)


def _wid():
    return lax.axis_index("s") * _NC + lax.axis_index("c")


def _start(src, pb, j, slotbuf, sem):
    plane = pb + (j >> 3)
    r0 = (j & 7) * _CR
    pltpu.async_copy(src.at[plane, pl.ds(r0, _CR)], slotbuf, sem)


def _wait(src, slotbuf, sem):
    pltpu.make_async_copy(src.at[0, pl.ds(0, _CR)], slotbuf, sem).wait()


def _pipeline(src, pb, buf, sem0, sem1, compute, init_carry):
    _start(src, pb, 0, buf.at[0], sem0)

    def pair_body(j, carry):
        _wait(src, buf.at[0], sem0)
        _start(src, pb, 2 * j + 1, buf.at[1], sem1)
        carry = compute(buf.at[0], carry)
        _wait(src, buf.at[1], sem1)

        @pl.when(j < _NCH // 2 - 1)
        def _():
            _start(src, pb, 2 * j + 2, buf.at[0], sem0)

        return compute(buf.at[1], carry)

    return lax.fori_loop(0, _NCH // 2, pair_body, init_carry, unroll=False)


_MMROWS = _PLANES * _P
_MMBLK = 1024
_MMG = _MMROWS // _MMBLK


def _minmax_tc_body(p_ref, t_ref, mn_ref, mx_ref):
    i = pl.program_id(0)
    y_mn = jnp.minimum(jnp.min(p_ref[...]), jnp.min(t_ref[...]))
    y_mx = jnp.maximum(jnp.max(p_ref[...]), jnp.max(t_ref[...]))

    @pl.when(i == 0)
    def _():
        mn_ref[0, 0] = y_mn
        mx_ref[0, 0] = y_mx

    @pl.when(i > 0)
    def _():
        mn_ref[0, 0] = jnp.minimum(mn_ref[0, 0], y_mn)
        mx_ref[0, 0] = jnp.maximum(mx_ref[0, 0], y_mx)


_minmax_tc = pl.pallas_call(
    _minmax_tc_body,
    grid=(_MMG,),
    in_specs=[
        pl.BlockSpec((_MMBLK, _P), lambda i: (i, 0)),
        pl.BlockSpec((_MMBLK, _P), lambda i: (i, 0)),
    ],
    out_specs=[
        pl.BlockSpec((1, 1), lambda i: (0, 0), memory_space=pltpu.SMEM),
        pl.BlockSpec((1, 1), lambda i: (0, 0), memory_space=pltpu.SMEM),
    ],
    out_shape=[
        jax.ShapeDtypeStruct((1, 1), jnp.float32),
        jax.ShapeDtypeStruct((1, 1), jnp.float32),
    ],
)


@functools.partial(
    pl.kernel,
    mesh=_mesh,
    out_type=jax.ShapeDtypeStruct((_NW, _TBINS * _L), jnp.float32),
    scratch_types=[
        pltpu.VMEM((2, _CR, _P), jnp.float32),
        pltpu.VMEM((_TBINS * _L,), jnp.float32),
        pltpu.VMEM((2 * _L,), jnp.float32),
        pltpu.SemaphoreType.DMA,
        pltpu.SemaphoreType.DMA,
    ],
    compiler_params=_cparams,
)
def _hist_k(p_hbm, t_hbm, par_hbm, out, buf, hist, par, sem0, sem1):
    w = _wid()
    pb = w * _PPT
    pltpu.sync_copy(par_hbm, par)
    lo = par[pl.ds(0, _L)]
    width = par[pl.ds(_L, _L)]
    lane = lax.iota(jnp.int32, _L)

    def zero_body(i, _):
        hist[pl.ds(i * _L, _L)] = jnp.zeros((_L,), jnp.float32)
        return 0

    lax.fori_loop(0, _TBINS, zero_body, 0)

    def make_compute(sign):
        valv = jnp.full((_L,), sign, jnp.float32)

        def compute(slab, carry):
            @plsc.parallel_loop(0, _CR, 1, unroll=1)
            def row_body(r):
                for g in range(_VPR // _GRP):
                    xs = [
                        slab[r, pl.ds((g * _GRP + u) * _L, _L)]
                        for u in range(_GRP)
                    ]
                    ss = [(x - lo) / width * 100.0 for x in xs]
                    idxs = [s.astype(jnp.int32) for s in ss]
                    addrs = [v * _L + lane for v in idxs]
                    masks = [x >= lo for x in xs]
                    for u in range(_GRP):
                        plsc.addupdate_scatter(
                            hist, [addrs[u]], valv, mask=masks[u]
                        )

            return carry

        return compute

    _pipeline(p_hbm, pb, buf, sem0, sem1, make_compute(1.0), 0)
    _pipeline(t_hbm, pb, buf, sem0, sem1, make_compute(-1.0), 0)
    pltpu.sync_copy(hist, out.at[w])


def kernel(prediction, target):
    p = prediction.reshape(_PLANES, _P, _P)
    t = target.reshape(_PLANES, _P, _P)
    mn, mx = _minmax_tc(
        prediction.reshape(_MMROWS, _P), target.reshape(_MMROWS, _P)
    )
    lo = mn[0, 0] + 0.1
    width = mx[0, 0] - lo
    params = jnp.concatenate(
        [jnp.full((_L,), lo, jnp.float32), jnp.full((_L,), width, jnp.float32)]
    )
    parts = _hist_k(p, t, params)
    diff = parts.reshape(_NW, _TBINS, _L).sum(axis=(0, 2))
    diff = diff[:_BINS].at[_BINS - 1].add(diff[_BINS])
    return jnp.mean(jnp.abs(diff))
